# Initial kernel scaffold; baseline (speedup 1.0000x reference)
import jax
import jax.numpy as jnp
from jax.experimental import pallas as pl


def kernel(hidden_states, router_weight, e_score_correction_bias, w_gate, w_up, w_down):
    raise NotImplementedError("write your pallas kernel here")


# speedup vs baseline: 2.8220x; 2.8220x over previous
import functools

import jax
import jax.numpy as jnp
from jax import lax
from jax.experimental import pallas as pl
from jax.experimental.pallas import tpu as pltpu

T = 2048
HIDDEN = 768
DFF = 512
N_ROUTED = 8
N_TOTAL = 72
NPAD = 128
SCALE = 2.5
NEG = -1e30


def _router_body(x_ref, rw_ref, bias_ref, e1_ref, e2_ref, w1_ref, w2_ref, zw_ref):
    x = x_ref[...]
    logits = lax.dot_general(x, rw_ref[...], (((1,), (1,)), ((), ())),
                             preferred_element_type=jnp.float32)
    col = lax.broadcasted_iota(jnp.int32, (T, NPAD), 1)
    valid = col < N_TOTAL
    logits = jnp.where(valid, logits, NEG)
    m = jnp.max(logits, axis=1, keepdims=True)
    p = jnp.exp(logits - m)
    p = jnp.where(valid, p, 0.0)
    scores = p / jnp.sum(p, axis=1, keepdims=True)
    sfc = jnp.where(valid, scores + bias_ref[...], NEG)

    m1 = jnp.max(sfc, axis=1, keepdims=True)
    i1 = jnp.min(jnp.where(sfc == m1, col, NPAD), axis=1, keepdims=True)
    sfc2 = jnp.where(col == i1, NEG, sfc)
    m2 = jnp.max(sfc2, axis=1, keepdims=True)
    i2 = jnp.min(jnp.where(sfc2 == m2, col, NPAD), axis=1, keepdims=True)

    s1 = jnp.sum(jnp.where(col == i1, scores, 0.0), axis=1, keepdims=True) * SCALE
    s2 = jnp.sum(jnp.where(col == i2, scores, 0.0), axis=1, keepdims=True) * SCALE

    z1 = i1 >= N_ROUTED
    z2 = i2 >= N_ROUTED
    e1_ref[...] = jnp.where(z1, -1, i1)
    e2_ref[...] = jnp.where(z2, -1, i2)
    w1_ref[...] = jnp.where(z1, 0.0, s1)
    w2_ref[...] = jnp.where(z2, 0.0, s2)
    zw_ref[...] = jnp.where(z1, s1, 0.0) + jnp.where(z2, s2, 0.0)


def _router(x, rw_pad, bias_pad):
    v = jax.ShapeDtypeStruct((T, 1), jnp.float32)
    iv = jax.ShapeDtypeStruct((T, 1), jnp.int32)
    return pl.pallas_call(
        _router_body,
        out_shape=(iv, iv, v, v, v),
    )(x, rw_pad, bias_pad)


def _expert_body(x_ref, wg_ref, wu_ref, wd_ref, e1_ref, e2_ref, w1_ref,
                 w2_ref, zw_ref, out_ref):
    e = pl.program_id(0)

    @pl.when(e == 0)
    def _():
        out_ref[...] = x_ref[...] * zw_ref[...]

    x = x_ref[...]
    g = jnp.dot(x, wg_ref[0], preferred_element_type=jnp.float32)
    u = jnp.dot(x, wu_ref[0], preferred_element_type=jnp.float32)
    h = g * jax.lax.logistic(g) * u
    y = jnp.dot(h, wd_ref[0], preferred_element_type=jnp.float32)
    w_tok = (jnp.where(e1_ref[...] == e, w1_ref[...], 0.0)
             + jnp.where(e2_ref[...] == e, w2_ref[...], 0.0))
    out_ref[...] += w_tok * y


def _experts(x, w_gate, w_up, w_down, e1, e2, w1, w2, zw):
    vspec = pl.BlockSpec((T, 1), lambda e: (0, 0))
    return pl.pallas_call(
        _expert_body,
        grid=(N_ROUTED,),
        in_specs=[
            pl.BlockSpec((T, HIDDEN), lambda e: (0, 0)),
            pl.BlockSpec((1, HIDDEN, DFF), lambda e: (e, 0, 0)),
            pl.BlockSpec((1, HIDDEN, DFF), lambda e: (e, 0, 0)),
            pl.BlockSpec((1, DFF, HIDDEN), lambda e: (e, 0, 0)),
            vspec, vspec, vspec, vspec, vspec,
        ],
        out_specs=pl.BlockSpec((T, HIDDEN), lambda e: (0, 0)),
        out_shape=jax.ShapeDtypeStruct((T, HIDDEN), jnp.float32),
    )(x, w_gate, w_up, w_down, e1, e2, w1, w2, zw)


def kernel(hidden_states, router_weight, e_score_correction_bias, w_gate, w_up, w_down):
    rw_pad = jnp.zeros((NPAD, HIDDEN), jnp.float32).at[:N_TOTAL].set(router_weight)
    bias_pad = jnp.full((1, NPAD), NEG, jnp.float32).at[0, :N_TOTAL].set(
        e_score_correction_bias)
    e1, e2, w1, w2, zw = _router(hidden_states, rw_pad, bias_pad)
    return _experts(hidden_states, w_gate, w_up, w_down, e1, e2, w1, w2, zw)
